# baseline (speedup 1.0000x reference)
import functools

import jax
import jax.numpy as jnp
from jax import lax
from jax.experimental import pallas as pl
from jax.experimental.pallas import tpu as pltpu
from jax.experimental.pallas import tpu_sc as plsc

NUM_CLASSES = 64
D = 128
N = 320000
NC, NS = 2, 16
NW = NC * NS
BLK = 128
NB = N // BLK
ITERS = (NB + NW - 1) // NW


def _sc_body(feat_hbm, lab_hbm, ones_hbm, zsum_hbm, sums_out, cnts_out,
             fblk2, labs_all, ones_v, acc_sh, cnt_sh, gsem, ssem):
  cid = lax.axis_index("c")
  sid = lax.axis_index("s")
  wid = sid * NC + cid

  def gather_start(bid, b):
    pltpu.async_copy(feat_hbm.at[bid], fblk2.at[b], gsem)

  def gather_wait(b):
    pltpu.make_async_copy(feat_hbm.at[0], fblk2.at[b], gsem).wait()

  def scatter_start(j, b):
    lab_row = labs_all.at[j]
    pltpu.async_copy(fblk2.at[b], acc_sh.at[lab_row], ssem, add=True)
    pltpu.async_copy(ones_v, cnt_sh.at[lab_row], ssem, add=True)

  def scatter_wait():
    pltpu.make_async_copy(fblk2.at[0], acc_sh.at[labs_all.at[0]], ssem).wait()
    pltpu.make_async_copy(ones_v, cnt_sh.at[labs_all.at[0]], ssem).wait()

  gather_start(wid, 0)

  @pl.when(sid == 0)
  def _():
    pltpu.sync_copy(zsum_hbm, acc_sh)
    pltpu.sync_copy(zsum_hbm, cnt_sh)

  pltpu.sync_copy(ones_hbm, ones_v)
  pltpu.sync_copy(lab_hbm.at[wid], labs_all)
  plsc.subcore_barrier()

  @pl.loop(0, ITERS, step=2)
  def _(j0):
    for b in range(2):
      j = j0 + b
      bid = j * NW + wid

      @pl.when(bid < NB)
      def _():
        gather_wait(b)

        @pl.when(j >= 1)
        def _():
          scatter_wait()

        @pl.when(bid + NW < NB)
        def _():
          gather_start(bid + NW, 1 - b)

        scatter_start(j, b)

  scatter_wait()
  plsc.subcore_barrier()

  @pl.when(sid == 0)
  def _():
    pltpu.sync_copy(acc_sh, sums_out.at[cid])
    pltpu.sync_copy(cnt_sh, cnts_out.at[cid])


_sc_segment_sums = functools.partial(
    pl.kernel,
    out_type=(
        jax.ShapeDtypeStruct((NC, NUM_CLASSES, D), jnp.float32),
        jax.ShapeDtypeStruct((NC, NUM_CLASSES, D), jnp.float32),
    ),
    mesh=plsc.VectorSubcoreMesh(core_axis_name="c", subcore_axis_name="s",
                                num_cores=NC, num_subcores=NS),
    scratch_types=[
        pltpu.VMEM((2, BLK, D), jnp.float32),
        pltpu.VMEM((ITERS, BLK), jnp.int32),
        pltpu.VMEM((BLK, D), jnp.float32),
        pltpu.VMEM_SHARED((NUM_CLASSES, D), jnp.float32),
        pltpu.VMEM_SHARED((NUM_CLASSES, D), jnp.float32),
        pltpu.SemaphoreType.DMA,
        pltpu.SemaphoreType.DMA,
    ],
)(_sc_body)


def _combine_body(sums_ref, cnts_ref, out_ref):
  s = sums_ref[0] + sums_ref[1]
  c = cnts_ref[0] + cnts_ref[1]
  out_ref[...] = s / c[:, 0:1]


def kernel(support_features, support_labels):
  feat = support_features.reshape(NB, BLK, D)
  lab = support_labels.astype(jnp.int32).reshape(NB, BLK)
  lab_rr = jnp.pad(lab, ((0, ITERS * NW - NB), (0, 0)))
  lab_rr = lab_rr.reshape(ITERS, NW, BLK).transpose(1, 0, 2)
  zsum = jnp.zeros((NUM_CLASSES, D), jnp.float32)
  ones = jnp.ones((BLK, D), jnp.float32)

  sums, cnts = _sc_segment_sums(feat, lab_rr, ones, zsum)

  return pl.pallas_call(
      _combine_body,
      out_shape=jax.ShapeDtypeStruct((NUM_CLASSES, D), jnp.float32),
  )(sums, cnts)


# speedup vs baseline: 1.7020x; 1.7020x over previous
import functools

import jax
import jax.numpy as jnp
from jax import lax
from jax.experimental import pallas as pl
from jax.experimental.pallas import tpu as pltpu
from jax.experimental.pallas import tpu_sc as plsc

NUM_CLASSES = 64
D = 128
N = 320000
NC, NS = 2, 16
NW = NC * NS
BLK = 128
NB = N // BLK
ITERS = (NB + NW - 1) // NW
SAMP = ((NB + 16 + 7) // 8) * 8


def _count_below(lblk, c):
  def step(_, carry):
    lo, hi = carry
    mid = (lo + hi) // 2
    pred = lblk[pl.ds(mid, 16)][0] < c
    return jnp.where(pred, mid + 1, lo), jnp.where(pred, hi, mid)

  _, hi = lax.fori_loop(0, 7, step, (jnp.int32(0), jnp.int32(BLK)))
  return hi


def _pos_of_class(samp_v, lab_blocks_hbm, lblk, c):
  def step(_, carry):
    lo, hi = carry
    mid = (lo + hi) // 2
    pred = samp_v[pl.ds(mid, 16)][0] < c
    return jnp.where(pred, mid + 1, lo), jnp.where(pred, hi, mid)

  lo, hi = lax.fori_loop(0, 12, step, (jnp.int32(0), jnp.int32(NB)))
  b = jnp.maximum(hi - 1, 0)
  pltpu.sync_copy(lab_blocks_hbm.at[b], lblk.at[pl.ds(0, BLK)])
  return (b * BLK + _count_below(lblk, c)).astype(jnp.float32)


def _sc_body(feat_hbm, lab_hbm, lab_blocks_hbm, samp_hbm, zsum_hbm,
             sums_out, cnts_out, fblk, lblk, labs_all, samp_v, cvec,
             acc_sh):
  cid = lax.axis_index("c")
  sid = lax.axis_index("s")
  wid = sid * NC + cid

  @pl.when(sid == 0)
  def _():
    pltpu.sync_copy(zsum_hbm, acc_sh)

  pltpu.sync_copy(lab_hbm.at[wid], labs_all)
  pltpu.sync_copy(samp_hbm, samp_v)

  c0 = 2 * wid
  p0 = _pos_of_class(samp_v, lab_blocks_hbm, lblk, c0)
  p1 = _pos_of_class(samp_v, lab_blocks_hbm, lblk, c0 + 1)
  p2 = _pos_of_class(samp_v, lab_blocks_hbm, lblk, c0 + 2)
  lane = lax.iota(jnp.int32, 16)
  cnts = jnp.where(lane == 0, p1 - p0, jnp.where(lane == 1, p2 - p1, 0.0))
  cvec[...] = cnts
  pltpu.sync_copy(cvec, cnts_out.at[wid])

  plsc.subcore_barrier()

  @pl.loop(0, ITERS)
  def _(j):
    bid = j * NW + wid

    @pl.when(bid < NB)
    def _():
      pltpu.sync_copy(feat_hbm.at[bid], fblk)
      pltpu.sync_copy(fblk, acc_sh.at[labs_all.at[j]], add=True)

  plsc.subcore_barrier()

  @pl.when(sid == 0)
  def _():
    pltpu.sync_copy(acc_sh, sums_out.at[cid])


_sc_segment_sums = functools.partial(
    pl.kernel,
    out_type=(
        jax.ShapeDtypeStruct((NC, NUM_CLASSES, D), jnp.float32),
        jax.ShapeDtypeStruct((NW, 16), jnp.float32),
    ),
    mesh=plsc.VectorSubcoreMesh(core_axis_name="c", subcore_axis_name="s",
                                num_cores=NC, num_subcores=NS),
    scratch_types=[
        pltpu.VMEM((BLK, D), jnp.float32),
        pltpu.VMEM((BLK + 16,), jnp.int32),
        pltpu.VMEM((ITERS, BLK), jnp.int32),
        pltpu.VMEM((SAMP,), jnp.int32),
        pltpu.VMEM((16,), jnp.float32),
        pltpu.VMEM_SHARED((NUM_CLASSES, D), jnp.float32),
    ],
)(_sc_body)


def _combine_body(sums_ref, cnts_ref, out_ref):
  s = sums_ref[0] + sums_ref[1]
  out_ref[...] = s / cnts_ref[...]


def kernel(support_features, support_labels):
  feat = support_features.reshape(NB, BLK, D)
  lab = support_labels.astype(jnp.int32).reshape(NB, BLK)
  lab_rr = jnp.pad(lab, ((0, ITERS * NW - NB), (0, 0)))
  lab_rr = lab_rr.reshape(ITERS, NW, BLK).transpose(1, 0, 2)
  samp = jnp.pad(lab[:, 0], (0, SAMP - NB))
  zsum = jnp.zeros((NUM_CLASSES, D), jnp.float32)

  sums, cnts = _sc_segment_sums(feat, lab_rr, lab, samp, zsum)
  counts_col = cnts[:, :2].reshape(NUM_CLASSES, 1)

  return pl.pallas_call(
      _combine_body,
      out_shape=jax.ShapeDtypeStruct((NUM_CLASSES, D), jnp.float32),
  )(sums, counts_col)


# speedup vs baseline: 2.4352x; 1.4308x over previous
import functools

import jax
import jax.numpy as jnp
from jax import lax
from jax.experimental import pallas as pl
from jax.experimental.pallas import tpu as pltpu
from jax.experimental.pallas import tpu_sc as plsc

NUM_CLASSES = 64
D = 128
N = 320000
NC, NS = 2, 16
NW = NC * NS
BLK = 128
NB = N // BLK
ITERS = (NB + NW - 1) // NW
SAMP = ((NB + 16 + 7) // 8) * 8
LPAD = BLK + 16


def _first_ge1(lref, c):
  def step(_, carry):
    lo, hi = carry
    mid = (lo + hi) // 2
    pred = lref[pl.ds(mid, 16)][0] < c
    return jnp.where(pred, mid + 1, lo), jnp.where(pred, hi, mid)

  _, hi = lax.fori_loop(0, 7, step, (jnp.int32(0), jnp.int32(BLK)))
  return hi


def _pos_of_class(samp_v, lab_blocks_hbm, lblk, c):
  def step(_, carry):
    lo, hi = carry
    mid = (lo + hi) // 2
    pred = samp_v[pl.ds(mid, 16)][0] < c
    return jnp.where(pred, mid + 1, lo), jnp.where(pred, hi, mid)

  lo, hi = lax.fori_loop(0, 12, step, (jnp.int32(0), jnp.int32(NB)))
  b = jnp.maximum(hi - 1, 0)
  pltpu.sync_copy(lab_blocks_hbm.at[b], lblk.at[pl.ds(0, BLK)])
  return (b * BLK + _first_ge1(lblk, c)).astype(jnp.float32)


def _sc_body(feat_hbm, lab_blocks_hbm, samp_hbm,
             sums_out, cnts_out, fblk2, lb_a, lb_b, lblk, samp_v, cvec,
             acc2, gsem):
  cid = lax.axis_index("c")
  sid = lax.axis_index("s")
  wid = sid * NC + cid
  lbufs = (lb_a, lb_b)

  def gather_start(bid, b):
    pltpu.async_copy(feat_hbm.at[bid], fblk2.at[b], gsem)
    pltpu.async_copy(lab_blocks_hbm.at[bid], lbufs[b].at[pl.ds(0, BLK)],
                     gsem)

  def gather_wait(b):
    pltpu.make_async_copy(feat_hbm.at[0], fblk2.at[b], gsem).wait()
    pltpu.make_async_copy(lab_blocks_hbm.at[0], lbufs[b].at[pl.ds(0, BLK)],
                          gsem).wait()

  gather_start(wid, 0)

  pltpu.sync_copy(samp_hbm, samp_v)

  c0 = 2 * wid
  p0 = _pos_of_class(samp_v, lab_blocks_hbm, lblk, c0)
  p1 = _pos_of_class(samp_v, lab_blocks_hbm, lblk, c0 + 1)
  p2 = _pos_of_class(samp_v, lab_blocks_hbm, lblk, c0 + 2)
  lane = lax.iota(jnp.int32, 16)
  cnts = jnp.where(lane == 0, p1 - p0, jnp.where(lane == 1, p2 - p1, 0.0))
  cvec[...] = cnts
  pltpu.sync_copy(cvec, cnts_out.at[wid])

  zeros16 = jnp.zeros((16,), jnp.float32)

  @pl.loop(0, NUM_CLASSES)
  def _(i):
    for k in range(D // 16):
      acc2[i, pl.ds(k * 16, 16)] = zeros16

  @pl.loop(0, ITERS, step=2)
  def _(j0):
    for b in range(2):
      j = j0 + b
      bid = j * NW + wid

      @pl.when(bid < NB)
      def _():
        gather_wait(b)

        @pl.when(bid + NW < NB)
        def _():
          gather_start(bid + NW, 1 - b)

        lb = lbufs[b]
        first = lb[pl.ds(0, 16)][0]
        last = lb[pl.ds(BLK - 1, 16)][0]

        def per_class(t, s):
          cls = first + t
          e = _first_ge1(lb, cls + 1)

          def per_row(r, vs):
            return tuple(
                vs[k] + fblk2[b, r, pl.ds(k * 16, 16)]
                for k in range(D // 16))

          vs = lax.fori_loop(
              s, e, per_row,
              tuple(jnp.zeros((16,), jnp.float32) for _ in range(D // 16)))
          for k in range(D // 16):
            acc2[cls, pl.ds(k * 16, 16)] = (
                acc2[cls, pl.ds(k * 16, 16)] + vs[k])
          return e

        lax.fori_loop(0, last - first + 1, per_class, jnp.int32(0))

  pltpu.sync_copy(acc2, sums_out.at[wid])


_sc_segment_sums = functools.partial(
    pl.kernel,
    out_type=(
        jax.ShapeDtypeStruct((NW, NUM_CLASSES, D), jnp.float32),
        jax.ShapeDtypeStruct((NW, 16), jnp.float32),
    ),
    mesh=plsc.VectorSubcoreMesh(core_axis_name="c", subcore_axis_name="s",
                                num_cores=NC, num_subcores=NS),
    scratch_types=[
        pltpu.VMEM((2, BLK, D), jnp.float32),
        pltpu.VMEM((LPAD,), jnp.int32),
        pltpu.VMEM((LPAD,), jnp.int32),
        pltpu.VMEM((LPAD,), jnp.int32),
        pltpu.VMEM((SAMP,), jnp.int32),
        pltpu.VMEM((16,), jnp.float32),
        pltpu.VMEM((NUM_CLASSES, D), jnp.float32),
        pltpu.SemaphoreType.DMA,
    ],
)(_sc_body)


def _combine_body(sums_ref, cnts_ref, out_ref):
  s = jnp.sum(sums_ref[...], axis=0)
  out_ref[...] = s / cnts_ref[...]


def kernel(support_features, support_labels):
  feat = support_features.reshape(NB, BLK, D)
  lab = support_labels.astype(jnp.int32).reshape(NB, BLK)
  samp = jnp.pad(lab[:, 0], (0, SAMP - NB))

  sums, cnts = _sc_segment_sums(feat, lab, samp)
  counts_col = cnts[:, :2].reshape(NUM_CLASSES, 1)

  return pl.pallas_call(
      _combine_body,
      out_shape=jax.ShapeDtypeStruct((NUM_CLASSES, D), jnp.float32),
  )(sums, counts_col)
